# Initial kernel scaffold; baseline (speedup 1.0000x reference)
import jax
import jax.numpy as jnp
from jax.experimental import pallas as pl


def kernel(x_enc, revin_w, revin_b, W_emb, b_emb, W_tp, b_tp, Wg_var, bg_var, Wf1_var, bf1_var, Wf2_var, bf2_var, Wg_t, bg_t, Wf1_t, bf1_t, Wf2_t, bf2_t, W_to, b_to, Wg1, bg1, Wg2, bg2, Wu1, bu1, Wu2, bu2, W_h, b_h):
    raise NotImplementedError("write your pallas kernel here")


# speedup vs baseline: 2.3317x; 2.3317x over previous
import functools
import jax
import jax.numpy as jnp
from jax.experimental import pallas as pl

B, L, N, D, DF, P, PL, H, K, EL = 16, 336, 321, 512, 512, 7, 48, 96, 3, 2
NEG = -1e9


def _ln(x):
    mu = x.mean(axis=-1, keepdims=True)
    var = x.var(axis=-1, keepdims=True)
    return (x - mu) / jnp.sqrt(var + 1e-5)


def _topk3_mask(S, axis):
    cols = jax.lax.broadcasted_iota(jnp.int32, S.shape, axis)
    Sm = S
    mask = jnp.zeros(S.shape, jnp.bool_)
    for _ in range(K):
        m = jnp.max(Sm, axis=axis, keepdims=True)
        c = jnp.min(jnp.where(Sm >= m, cols, jnp.int32(2**30)), axis=axis,
                    keepdims=True)
        hit = cols == c
        mask = jnp.logical_or(mask, hit)
        Sm = jnp.where(hit, NEG, Sm)
    return mask


def _softmax(x, axis=-1):
    m = jnp.max(x, axis=axis, keepdims=True)
    e = jnp.exp(x - m)
    return e / jnp.sum(e, axis=axis, keepdims=True)


def _prep_body(x_ref, w_ref, b_ref, xn_ref, mean_ref, std_ref):
    x = x_ref[0]
    mean = jnp.mean(x, axis=0, keepdims=True)
    var = jnp.mean((x - mean) * (x - mean), axis=0, keepdims=True)
    std = jnp.sqrt(var + 1e-5)
    xn_ref[0] = (x - mean) / std * w_ref[...] + b_ref[...]
    mean_ref[0] = mean
    std_ref[0] = std


def _embed_var_body(xt_ref, We_ref, be_ref, tok_ref, av_ref):
    xt = xt_ref[0]
    tok_ref[0] = jnp.dot(xt, We_ref[...],
                         preferred_element_type=jnp.float32) + be_ref[...]
    nrm = jnp.sqrt(jnp.sum(xt * xt, axis=-1, keepdims=True))
    xu = xt / (nrm + 1e-8)
    S = jax.lax.dot_general(xu, xu, (((1,), (1,)), ((), ())),
                            preferred_element_type=jnp.float32)
    mask = _topk3_mask(S, axis=1)
    av_ref[0] = _softmax(jnp.where(mask, S, NEG), axis=-1)


def _gcn_body(tok_ref, A_ref, Wg_ref, bg_ref, W1_ref, b1_ref, W2_ref, b2_ref,
              out_ref):
    h = tok_ref[0]
    A = A_ref[0]
    for l in range(EL):
        m = jnp.dot(A, h, preferred_element_type=jnp.float32)
        g = jax.nn.gelu(jnp.dot(m, Wg_ref[l],
                                preferred_element_type=jnp.float32)
                        + bg_ref[l][None, :])
        h = _ln(h + g)
        f = jax.nn.gelu(jnp.dot(h, W1_ref[l],
                                preferred_element_type=jnp.float32)
                        + b1_ref[l][None, :])
        f = jnp.dot(f, W2_ref[l], preferred_element_type=jnp.float32) \
            + b2_ref[l][None, :]
        h = _ln(h + f)
    out_ref[0] = h


def _embed_time_body(xp_ref, Wt_ref, bt_ref, tpt_ref, tu_ref):
    xp = xp_ref[0]
    tpt = jnp.dot(xp, Wt_ref[...],
                  preferred_element_type=jnp.float32) + bt_ref[...]
    tpt_ref[0] = tpt
    nrm = jnp.sqrt(jnp.sum(tpt * tpt, axis=-1, keepdims=True))
    tu_ref[0] = tpt / (nrm + 1e-8)


def _adj_time_body(tu_ref, at_ref):
    tu = tu_ref[0]
    cols = []
    for q in range(P):
        prod = tu * tu[:, q:q + 1, :]
        cols.append(jnp.sum(prod, axis=-1, keepdims=True))
    St = jnp.concatenate(cols, axis=-1)
    mask = _topk3_mask(St, axis=2)
    at_ref[0] = _softmax(jnp.where(mask, St, NEG), axis=-1)


def _msg_time_body(at_ref, h_ref, m_ref):
    At = at_ref[0]
    h = h_ref[0]
    m = At[:, :, 0:1] * h[:, 0:1, :]
    for q in range(1, P):
        m = m + At[:, :, q:q + 1] * h[:, q:q + 1, :]
    m_ref[0] = m


def _dense_time_body(l, m_ref, h_ref, Wg_ref, bg_ref, W1_ref, b1_ref,
                     W2_ref, b2_ref, out_ref):
    m = m_ref[0]
    h = h_ref[0]
    g = jax.nn.gelu(jnp.dot(m, Wg_ref[l],
                            preferred_element_type=jnp.float32)
                    + bg_ref[l][None, :])
    h = _ln(h + g)
    f = jax.nn.gelu(jnp.dot(h, W1_ref[l],
                            preferred_element_type=jnp.float32)
                    + b1_ref[l][None, :])
    f = jnp.dot(f, W2_ref[l], preferred_element_type=jnp.float32) \
        + b2_ref[l][None, :]
    out_ref[0] = _ln(h + f)


def _ztime_body(h_ref, Wt_ref, bt_ref, out_ref):
    h = h_ref[0]
    zm = jnp.mean(h, axis=1)
    out_ref[0] = jnp.dot(zm, Wt_ref[...],
                         preferred_element_type=jnp.float32) + bt_ref[...]


def _fusion_body(zv_ref, zt_ref, tok_ref, Wg1a_ref, Wg1b_ref, bg1_ref,
                 Wg2_ref, bg2_ref, Wu1a_ref, Wu1b_ref, bu1_ref, Wu2_ref,
                 bu2_ref, Wh_ref, bh_ref, rw_ref, rb_ref, mean_ref, std_ref,
                 out_ref):
    zv = zv_ref[0]
    zt = zt_ref[0]
    g1 = jax.nn.gelu(jnp.dot(zv, Wg1a_ref[...],
                             preferred_element_type=jnp.float32)
                     + jnp.dot(zt, Wg1b_ref[...],
                               preferred_element_type=jnp.float32)
                     + bg1_ref[...])
    wl = jnp.dot(g1, Wg2_ref[...],
                 preferred_element_type=jnp.float32) + bg2_ref[...]
    wsm = _softmax(wl, axis=-1)
    u = jax.nn.gelu(jnp.dot(zv, Wu1a_ref[...],
                            preferred_element_type=jnp.float32)
                    + jnp.dot(zt, Wu1b_ref[...],
                              preferred_element_type=jnp.float32)
                    + bu1_ref[...])
    u = jnp.dot(u, Wu2_ref[...],
                preferred_element_type=jnp.float32) + bu2_ref[...]
    fused = wsm[:, 0:1] * zv + wsm[:, 1:2] * zt + 0.1 * u + 0.2 * tok_ref[0]
    y = jnp.dot(fused, Wh_ref[...],
                preferred_element_type=jnp.float32) + bh_ref[...]
    y = (y - rb_ref[...]) / (rw_ref[...] + 1e-10) * std_ref[0] + mean_ref[0]
    out_ref[0] = y


def _b3(shape):
    return pl.BlockSpec((1,) + shape, lambda b: (0, 0, 0))


def _full(shape):
    nd = len(shape)
    return pl.BlockSpec(shape, lambda b: (0,) * nd)


def _bat(shape):
    nd = len(shape) + 1
    zeros = (0,) * len(shape)
    return pl.BlockSpec((1,) + shape, lambda b, z=zeros: (b,) + z)


def kernel(x_enc, revin_w, revin_b, W_emb, b_emb, W_tp, b_tp, Wg_var, bg_var,
           Wf1_var, bf1_var, Wf2_var, bf2_var, Wg_t, bg_t, Wf1_t, bf1_t,
           Wf2_t, bf2_t, W_to, b_to, Wg1, bg1, Wg2, bg2, Wu1, bu1, Wu2, bu2,
           W_h, b_h):
    f32 = jnp.float32
    nb = x_enc.shape[0]
    grid = (nb,)

    xn, mean, std = pl.pallas_call(
        _prep_body,
        grid=grid,
        in_specs=[_bat((L, N)), _full((1, N)), _full((1, N))],
        out_specs=[_bat((L, N)), _bat((1, N)), _bat((1, N))],
        out_shape=[jax.ShapeDtypeStruct((nb, L, N), f32),
                   jax.ShapeDtypeStruct((nb, 1, N), f32),
                   jax.ShapeDtypeStruct((nb, 1, N), f32)],
    )(x_enc, revin_w.reshape(1, N), revin_b.reshape(1, N))

    xt = jnp.swapaxes(xn, 1, 2)
    xp = xt.reshape(nb, N * P, PL)

    tokens, Av = pl.pallas_call(
        _embed_var_body,
        grid=grid,
        in_specs=[_bat((N, L)), _full((L, D)), _full((1, D))],
        out_specs=[_bat((N, D)), _bat((N, N))],
        out_shape=[jax.ShapeDtypeStruct((nb, N, D), f32),
                   jax.ShapeDtypeStruct((nb, N, N), f32)],
    )(xt, W_emb, b_emb.reshape(1, D))

    z_var = pl.pallas_call(
        _gcn_body,
        grid=grid,
        in_specs=[_bat((N, D)), _bat((N, N)), _full((EL, D, D)),
                  _full((EL, D)), _full((EL, D, DF)), _full((EL, DF)),
                  _full((EL, DF, D)), _full((EL, D))],
        out_specs=_bat((N, D)),
        out_shape=jax.ShapeDtypeStruct((nb, N, D), f32),
    )(tokens, Av, Wg_var, bg_var, Wf1_var, bf1_var, Wf2_var, bf2_var)

    tpt, tu = pl.pallas_call(
        _embed_time_body,
        grid=grid,
        in_specs=[_bat((N * P, PL)), _full((PL, D)), _full((1, D))],
        out_specs=[_bat((N * P, D)), _bat((N * P, D))],
        out_shape=[jax.ShapeDtypeStruct((nb, N * P, D), f32),
                   jax.ShapeDtypeStruct((nb, N * P, D), f32)],
    )(xp, W_tp, b_tp.reshape(1, D))

    tu3 = tu.reshape(nb, N, P, D)
    At = pl.pallas_call(
        _adj_time_body,
        grid=grid,
        in_specs=[_bat((N, P, D))],
        out_specs=_bat((N, P, P)),
        out_shape=jax.ShapeDtypeStruct((nb, N, P, P), f32),
    )(tu3)

    h_t = tpt
    for l in range(EL):
        m3 = pl.pallas_call(
            _msg_time_body,
            grid=grid,
            in_specs=[_bat((N, P, P)), _bat((N, P, D))],
            out_specs=_bat((N, P, D)),
            out_shape=jax.ShapeDtypeStruct((nb, N, P, D), f32),
        )(At, h_t.reshape(nb, N, P, D))
        h_t = pl.pallas_call(
            functools.partial(_dense_time_body, l),
            grid=grid,
            in_specs=[_bat((N * P, D)), _bat((N * P, D)), _full((EL, D, D)),
                      _full((EL, D)), _full((EL, D, DF)), _full((EL, DF)),
                      _full((EL, DF, D)), _full((EL, D))],
            out_specs=_bat((N * P, D)),
            out_shape=jax.ShapeDtypeStruct((nb, N * P, D), f32),
        )(m3.reshape(nb, N * P, D), h_t, Wg_t, bg_t, Wf1_t, bf1_t,
          Wf2_t, bf2_t)

    z_time = pl.pallas_call(
        _ztime_body,
        grid=grid,
        in_specs=[_bat((N, P, D)), _full((D, D)), _full((1, D))],
        out_specs=_bat((N, D)),
        out_shape=jax.ShapeDtypeStruct((nb, N, D), f32),
    )(h_t.reshape(nb, N, P, D), W_to, b_to.reshape(1, D))

    stdT = jnp.swapaxes(std, 1, 2)
    meanT = jnp.swapaxes(mean, 1, 2)
    yk = pl.pallas_call(
        _fusion_body,
        grid=grid,
        in_specs=[_bat((N, D)), _bat((N, D)), _bat((N, D)),
                  _full((D, D)), _full((D, D)), _full((1, D)),
                  _full((D, 2)), _full((1, 2)),
                  _full((D, D)), _full((D, D)), _full((1, D)),
                  _full((D, D)), _full((1, D)),
                  _full((D, H)), _full((1, H)),
                  _full((N, 1)), _full((N, 1)),
                  _bat((N, 1)), _bat((N, 1))],
        out_specs=_bat((N, H)),
        out_shape=jax.ShapeDtypeStruct((nb, N, H), f32),
    )(z_var, z_time, tokens,
      Wg1[:D], Wg1[D:], bg1.reshape(1, D),
      Wg2, bg2.reshape(1, 2),
      Wu1[:D], Wu1[D:], bu1.reshape(1, D),
      Wu2, bu2.reshape(1, D),
      W_h, b_h.reshape(1, H),
      revin_w.reshape(N, 1), revin_b.reshape(N, 1),
      meanT, stdT)

    return jnp.swapaxes(yk, 1, 2)


# speedup vs baseline: 6.1666x; 2.6446x over previous
import jax
import jax.numpy as jnp
from jax.experimental import pallas as pl

B, L, N, D, DF, P, PL, H, K, EL = 16, 336, 321, 512, 512, 7, 48, 96, 3, 2
NEG = -1e9


def _ln(x):
    mu = x.mean(axis=-1, keepdims=True)
    var = x.var(axis=-1, keepdims=True)
    return (x - mu) / jnp.sqrt(var + 1e-5)


def _topk3_mask(S, axis):
    cols = jax.lax.broadcasted_iota(jnp.int32, S.shape, axis)
    Sm = S
    mask = jnp.zeros(S.shape, jnp.bool_)
    for _ in range(K):
        m = jnp.max(Sm, axis=axis, keepdims=True)
        c = jnp.min(jnp.where(Sm >= m, cols, jnp.int32(2**30)), axis=axis,
                    keepdims=True)
        hit = cols == c
        mask = jnp.logical_or(mask, hit)
        Sm = jnp.where(hit, NEG, Sm)
    return mask


def _softmax(x, axis=-1):
    m = jnp.max(x, axis=axis, keepdims=True)
    e = jnp.exp(x - m)
    return e / jnp.sum(e, axis=axis, keepdims=True)


def _dot(a, b):
    return jnp.dot(a, b, preferred_element_type=jnp.float32)


def _prep_body(x_ref, w_ref, b_ref, xn_ref, mean_ref, std_ref):
    x = x_ref[0]
    mean = jnp.mean(x, axis=0, keepdims=True)
    var = jnp.mean((x - mean) * (x - mean), axis=0, keepdims=True)
    std = jnp.sqrt(var + 1e-5)
    xn_ref[0] = (x - mean) / std * w_ref[...] + b_ref[...]
    mean_ref[0] = mean
    std_ref[0] = std


def _mega_body(xt_ref, We_ref, be_ref, Wt_ref, bt_ref,
               Wgv_ref, bgv_ref, W1v_ref, b1v_ref, W2v_ref, b2v_ref,
               Wgt_ref, bgt_ref, W1t_ref, b1t_ref, W2t_ref, b2t_ref,
               Wto_ref, bto_ref,
               Wg1a_ref, Wg1b_ref, bg1_ref, Wg2_ref, bg2_ref,
               Wu1a_ref, Wu1b_ref, bu1_ref, Wu2_ref, bu2_ref,
               Wh_ref, bh_ref, rw_ref, rb_ref, mean_ref, std_ref,
               out_ref):
    xt = xt_ref[0]

    tokens = _dot(xt, We_ref[...]) + be_ref[...]
    inv = 1.0 / (jnp.sqrt(jnp.sum(xt * xt, axis=-1, keepdims=True)) + 1e-8)
    S = jax.lax.dot_general(xt, xt, (((1,), (1,)), ((), ())),
                            preferred_element_type=jnp.float32)
    S = S * inv * inv.reshape(1, N)
    mask = _topk3_mask(S, axis=1)
    Av = _softmax(jnp.where(mask, S, NEG), axis=-1)
    h = tokens
    for l in range(EL):
        m = _dot(Av, h)
        g = jax.nn.gelu(_dot(m, Wgv_ref[l]) + bgv_ref[l][None, :])
        h = _ln(h + g)
        f = jax.nn.gelu(_dot(h, W1v_ref[l]) + b1v_ref[l][None, :])
        f = _dot(f, W2v_ref[l]) + b2v_ref[l][None, :]
        h = _ln(h + f)
    z_var = h

    hb = []
    invb = []
    for p in range(P):
        tp = _dot(xt[:, p * PL:(p + 1) * PL], Wt_ref[...]) + bt_ref[...]
        hb.append(tp)
        invb.append(1.0 / (jnp.sqrt(jnp.sum(tp * tp, axis=-1,
                                            keepdims=True)) + 1e-8))
    invrow = jnp.concatenate(invb, axis=1)
    Arow = []
    for p in range(P):
        sp = jnp.concatenate(
            [jnp.sum(hb[p] * hb[q], axis=-1, keepdims=True)
             for q in range(P)], axis=1)
        sp = sp * invrow * invb[p]
        mk = _topk3_mask(sp, axis=1)
        Arow.append(_softmax(jnp.where(mk, sp, NEG), axis=-1))
    for l in range(EL):
        mb = []
        for p in range(P):
            m = Arow[p][:, 0:1] * hb[0]
            for q in range(1, P):
                m = m + Arow[p][:, q:q + 1] * hb[q]
            mb.append(m)
        for p in range(P):
            g = jax.nn.gelu(_dot(mb[p], Wgt_ref[l]) + bgt_ref[l][None, :])
            hp = _ln(hb[p] + g)
            f = jax.nn.gelu(_dot(hp, W1t_ref[l]) + b1t_ref[l][None, :])
            f = _dot(f, W2t_ref[l]) + b2t_ref[l][None, :]
            hb[p] = _ln(hp + f)
    zm = hb[0]
    for p in range(1, P):
        zm = zm + hb[p]
    zm = zm * (1.0 / P)
    z_time = _dot(zm, Wto_ref[...]) + bto_ref[...]

    g1 = jax.nn.gelu(_dot(z_var, Wg1a_ref[...]) + _dot(z_time, Wg1b_ref[...])
                     + bg1_ref[...])
    wsm = _softmax(_dot(g1, Wg2_ref[...]) + bg2_ref[...], axis=-1)
    u = jax.nn.gelu(_dot(z_var, Wu1a_ref[...]) + _dot(z_time, Wu1b_ref[...])
                    + bu1_ref[...])
    u = _dot(u, Wu2_ref[...]) + bu2_ref[...]
    fused = wsm[:, 0:1] * z_var + wsm[:, 1:2] * z_time + 0.1 * u \
        + 0.2 * tokens
    y = _dot(fused, Wh_ref[...]) + bh_ref[...]
    out_ref[0] = (y - rb_ref[...]) / (rw_ref[...] + 1e-10) * std_ref[0] \
        + mean_ref[0]


def _full(shape):
    nd = len(shape)
    return pl.BlockSpec(shape, lambda b: (0,) * nd)


def _bat(shape):
    zeros = (0,) * len(shape)
    return pl.BlockSpec((1,) + shape, lambda b, z=zeros: (b,) + z)


def kernel(x_enc, revin_w, revin_b, W_emb, b_emb, W_tp, b_tp, Wg_var, bg_var,
           Wf1_var, bf1_var, Wf2_var, bf2_var, Wg_t, bg_t, Wf1_t, bf1_t,
           Wf2_t, bf2_t, W_to, b_to, Wg1, bg1, Wg2, bg2, Wu1, bu1, Wu2, bu2,
           W_h, b_h):
    f32 = jnp.float32
    nb = x_enc.shape[0]
    grid = (nb,)

    xn, mean, std = pl.pallas_call(
        _prep_body,
        grid=grid,
        in_specs=[_bat((L, N)), _full((1, N)), _full((1, N))],
        out_specs=[_bat((L, N)), _bat((1, N)), _bat((1, N))],
        out_shape=[jax.ShapeDtypeStruct((nb, L, N), f32),
                   jax.ShapeDtypeStruct((nb, 1, N), f32),
                   jax.ShapeDtypeStruct((nb, 1, N), f32)],
    )(x_enc, revin_w.reshape(1, N), revin_b.reshape(1, N))

    xt = jnp.swapaxes(xn, 1, 2)
    stdT = jnp.swapaxes(std, 1, 2)
    meanT = jnp.swapaxes(mean, 1, 2)

    yk = pl.pallas_call(
        _mega_body,
        grid=grid,
        in_specs=[_bat((N, L)), _full((L, D)), _full((1, D)),
                  _full((PL, D)), _full((1, D)),
                  _full((EL, D, D)), _full((EL, D)), _full((EL, D, DF)),
                  _full((EL, DF)), _full((EL, DF, D)), _full((EL, D)),
                  _full((EL, D, D)), _full((EL, D)), _full((EL, D, DF)),
                  _full((EL, DF)), _full((EL, DF, D)), _full((EL, D)),
                  _full((D, D)), _full((1, D)),
                  _full((D, D)), _full((D, D)), _full((1, D)),
                  _full((D, 2)), _full((1, 2)),
                  _full((D, D)), _full((D, D)), _full((1, D)),
                  _full((D, D)), _full((1, D)),
                  _full((D, H)), _full((1, H)),
                  _full((N, 1)), _full((N, 1)), _bat((N, 1)), _bat((N, 1))],
        out_specs=_bat((N, H)),
        out_shape=jax.ShapeDtypeStruct((nb, N, H), f32),
    )(xt, W_emb, b_emb.reshape(1, D), W_tp, b_tp.reshape(1, D),
      Wg_var, bg_var, Wf1_var, bf1_var, Wf2_var, bf2_var,
      Wg_t, bg_t, Wf1_t, bf1_t, Wf2_t, bf2_t,
      W_to, b_to.reshape(1, D),
      Wg1[:D], Wg1[D:], bg1.reshape(1, D), Wg2, bg2.reshape(1, 2),
      Wu1[:D], Wu1[D:], bu1.reshape(1, D), Wu2, bu2.reshape(1, D),
      W_h, b_h.reshape(1, H),
      revin_w.reshape(N, 1), revin_b.reshape(N, 1), meanT, stdT)

    return jnp.swapaxes(yk, 1, 2)


# speedup vs baseline: 6.2386x; 1.0117x over previous
import jax
import jax.numpy as jnp
from jax.experimental import pallas as pl

B, L, N, D, DF, P, PL, H, K, EL = 16, 336, 321, 512, 512, 7, 48, 96, 3, 2
NEG = -1e9


def _ln(x):
    mu = x.mean(axis=-1, keepdims=True)
    var = x.var(axis=-1, keepdims=True)
    return (x - mu) / jnp.sqrt(var + 1e-5)


def _topk3_mask(S, axis):
    cols = jax.lax.broadcasted_iota(jnp.int32, S.shape, axis)
    Sm = S
    mask = jnp.zeros(S.shape, jnp.bool_)
    for _ in range(K):
        m = jnp.max(Sm, axis=axis, keepdims=True)
        c = jnp.min(jnp.where(Sm >= m, cols, jnp.int32(2**30)), axis=axis,
                    keepdims=True)
        hit = cols == c
        mask = jnp.logical_or(mask, hit)
        Sm = jnp.where(hit, NEG, Sm)
    return mask


def _softmax(x, axis=-1):
    m = jnp.max(x, axis=axis, keepdims=True)
    e = jnp.exp(x - m)
    return e / jnp.sum(e, axis=axis, keepdims=True)


def _dot(a, b):
    return jnp.dot(a, b, preferred_element_type=jnp.float32)


def _prep_body(x_ref, w_ref, b_ref, xn_ref, mean_ref, std_ref):
    x = x_ref[0]
    mean = jnp.mean(x, axis=0, keepdims=True)
    var = jnp.mean((x - mean) * (x - mean), axis=0, keepdims=True)
    std = jnp.sqrt(var + 1e-5)
    xn_ref[0] = (x - mean) / std * w_ref[...] + b_ref[...]
    mean_ref[0] = mean
    std_ref[0] = std


def _mega_body(xt_ref, We_ref, be_ref, Wt_ref, bt_ref,
               Wgv_ref, bgv_ref, W1v_ref, b1v_ref, W2v_ref, b2v_ref,
               Wgt_ref, bgt_ref, W1t_ref, b1t_ref, W2t_ref, b2t_ref,
               Wto_ref, bto_ref,
               Wg1a_ref, Wg1b_ref, bg1_ref, Wg2_ref, bg2_ref,
               Wu1a_ref, Wu1b_ref, bu1_ref, Wu2_ref, bu2_ref,
               Wh_ref, bh_ref, rw_ref, rb_ref, mean_ref, std_ref,
               out_ref):
    xt = xt_ref[0]

    tokens = _dot(xt, We_ref[...]) + be_ref[...]
    xu = xt / (jnp.sqrt(jnp.sum(xt * xt, axis=-1, keepdims=True)) + 1e-8)
    S = jax.lax.dot_general(xu, xu, (((1,), (1,)), ((), ())),
                            preferred_element_type=jnp.float32)
    mask = _topk3_mask(S, axis=1)
    Av = _softmax(jnp.where(mask, S, NEG), axis=-1)
    h = tokens
    for l in range(EL):
        m = _dot(Av, h)
        g = jax.nn.gelu(_dot(m, Wgv_ref[l]) + bgv_ref[l][None, :])
        h = _ln(h + g)
        f = jax.nn.gelu(_dot(h, W1v_ref[l]) + b1v_ref[l][None, :])
        f = _dot(f, W2v_ref[l]) + b2v_ref[l][None, :]
        h = _ln(h + f)
    z_var = h

    hb = []
    tub = []
    for p in range(P):
        tp = _dot(xt[:, p * PL:(p + 1) * PL], Wt_ref[...]) + bt_ref[...]
        hb.append(tp)
        tub.append(tp / (jnp.sqrt(jnp.sum(tp * tp, axis=-1,
                                          keepdims=True)) + 1e-8))
    Arow = []
    for p in range(P):
        sp = jnp.concatenate(
            [jnp.sum(tub[p] * tub[q], axis=-1, keepdims=True)
             for q in range(P)], axis=1)
        mk = _topk3_mask(sp, axis=1)
        Arow.append(_softmax(jnp.where(mk, sp, NEG), axis=-1))
    for l in range(EL):
        mb = []
        for p in range(P):
            m = Arow[p][:, 0:1] * hb[0]
            for q in range(1, P):
                m = m + Arow[p][:, q:q + 1] * hb[q]
            mb.append(m)
        for p in range(P):
            g = jax.nn.gelu(_dot(mb[p], Wgt_ref[l]) + bgt_ref[l][None, :])
            hp = _ln(hb[p] + g)
            f = jax.nn.gelu(_dot(hp, W1t_ref[l]) + b1t_ref[l][None, :])
            f = _dot(f, W2t_ref[l]) + b2t_ref[l][None, :]
            hb[p] = _ln(hp + f)
    zm = hb[0]
    for p in range(1, P):
        zm = zm + hb[p]
    zm = zm * (1.0 / P)
    z_time = _dot(zm, Wto_ref[...]) + bto_ref[...]

    g1 = jax.nn.gelu(_dot(z_var, Wg1a_ref[...]) + _dot(z_time, Wg1b_ref[...])
                     + bg1_ref[...])
    wsm = _softmax(_dot(g1, Wg2_ref[...]) + bg2_ref[...], axis=-1)
    u = jax.nn.gelu(_dot(z_var, Wu1a_ref[...]) + _dot(z_time, Wu1b_ref[...])
                    + bu1_ref[...])
    u = _dot(u, Wu2_ref[...]) + bu2_ref[...]
    fused = wsm[:, 0:1] * z_var + wsm[:, 1:2] * z_time + 0.1 * u \
        + 0.2 * tokens
    y = _dot(fused, Wh_ref[...]) + bh_ref[...]
    out_ref[0] = (y - rb_ref[...]) / (rw_ref[...] + 1e-10) * std_ref[0] \
        + mean_ref[0]


def _full(shape):
    nd = len(shape)
    return pl.BlockSpec(shape, lambda b: (0,) * nd)


def _bat(shape):
    zeros = (0,) * len(shape)
    return pl.BlockSpec((1,) + shape, lambda b, z=zeros: (b,) + z)


def kernel(x_enc, revin_w, revin_b, W_emb, b_emb, W_tp, b_tp, Wg_var, bg_var,
           Wf1_var, bf1_var, Wf2_var, bf2_var, Wg_t, bg_t, Wf1_t, bf1_t,
           Wf2_t, bf2_t, W_to, b_to, Wg1, bg1, Wg2, bg2, Wu1, bu1, Wu2, bu2,
           W_h, b_h):
    f32 = jnp.float32
    nb = x_enc.shape[0]
    grid = (nb,)

    xn, mean, std = pl.pallas_call(
        _prep_body,
        grid=grid,
        in_specs=[_bat((L, N)), _full((1, N)), _full((1, N))],
        out_specs=[_bat((L, N)), _bat((1, N)), _bat((1, N))],
        out_shape=[jax.ShapeDtypeStruct((nb, L, N), f32),
                   jax.ShapeDtypeStruct((nb, 1, N), f32),
                   jax.ShapeDtypeStruct((nb, 1, N), f32)],
    )(x_enc, revin_w.reshape(1, N), revin_b.reshape(1, N))

    xt = jnp.swapaxes(xn, 1, 2)
    stdT = jnp.swapaxes(std, 1, 2)
    meanT = jnp.swapaxes(mean, 1, 2)

    yk = pl.pallas_call(
        _mega_body,
        grid=grid,
        in_specs=[_bat((N, L)), _full((L, D)), _full((1, D)),
                  _full((PL, D)), _full((1, D)),
                  _full((EL, D, D)), _full((EL, D)), _full((EL, D, DF)),
                  _full((EL, DF)), _full((EL, DF, D)), _full((EL, D)),
                  _full((EL, D, D)), _full((EL, D)), _full((EL, D, DF)),
                  _full((EL, DF)), _full((EL, DF, D)), _full((EL, D)),
                  _full((D, D)), _full((1, D)),
                  _full((D, D)), _full((D, D)), _full((1, D)),
                  _full((D, 2)), _full((1, 2)),
                  _full((D, D)), _full((D, D)), _full((1, D)),
                  _full((D, D)), _full((1, D)),
                  _full((D, H)), _full((1, H)),
                  _full((N, 1)), _full((N, 1)), _bat((N, 1)), _bat((N, 1))],
        out_specs=_bat((N, H)),
        out_shape=jax.ShapeDtypeStruct((nb, N, H), f32),
    )(xt, W_emb, b_emb.reshape(1, D), W_tp, b_tp.reshape(1, D),
      Wg_var, bg_var, Wf1_var, bf1_var, Wf2_var, bf2_var,
      Wg_t, bg_t, Wf1_t, bf1_t, Wf2_t, bf2_t,
      W_to, b_to.reshape(1, D),
      Wg1[:D], Wg1[D:], bg1.reshape(1, D), Wg2, bg2.reshape(1, 2),
      Wu1[:D], Wu1[D:], bu1.reshape(1, D), Wu2, bu2.reshape(1, D),
      W_h, b_h.reshape(1, H),
      revin_w.reshape(N, 1), revin_b.reshape(N, 1), meanT, stdT)

    return jnp.swapaxes(yk, 1, 2)


# speedup vs baseline: 6.2659x; 1.0044x over previous
import jax
import jax.numpy as jnp
from jax.experimental import pallas as pl
from jax.experimental.pallas import tpu as pltpu

_PAR = pltpu.CompilerParams(dimension_semantics=("parallel",))

B, L, N, D, DF, P, PL, H, K, EL = 16, 336, 321, 512, 512, 7, 48, 96, 3, 2
NEG = -1e9


def _ln(x):
    mu = x.mean(axis=-1, keepdims=True)
    var = x.var(axis=-1, keepdims=True)
    return (x - mu) / jnp.sqrt(var + 1e-5)


def _topk3_mask(S, axis):
    cols = jax.lax.broadcasted_iota(jnp.int32, S.shape, axis)
    Sm = S
    mask = jnp.zeros(S.shape, jnp.bool_)
    for _ in range(K):
        m = jnp.max(Sm, axis=axis, keepdims=True)
        c = jnp.min(jnp.where(Sm >= m, cols, jnp.int32(2**30)), axis=axis,
                    keepdims=True)
        hit = cols == c
        mask = jnp.logical_or(mask, hit)
        Sm = jnp.where(hit, NEG, Sm)
    return mask


def _softmax(x, axis=-1):
    m = jnp.max(x, axis=axis, keepdims=True)
    e = jnp.exp(x - m)
    return e / jnp.sum(e, axis=axis, keepdims=True)


def _dot(a, b):
    return jnp.dot(a, b, preferred_element_type=jnp.float32)


def _prep_body(x_ref, w_ref, b_ref, xn_ref, mean_ref, std_ref):
    x = x_ref[0]
    mean = jnp.mean(x, axis=0, keepdims=True)
    var = jnp.mean((x - mean) * (x - mean), axis=0, keepdims=True)
    std = jnp.sqrt(var + 1e-5)
    xn_ref[0] = (x - mean) / std * w_ref[...] + b_ref[...]
    mean_ref[0] = mean
    std_ref[0] = std


def _mega_body(xt_ref, We_ref, be_ref, Wt_ref, bt_ref,
               Wgv_ref, bgv_ref, W1v_ref, b1v_ref, W2v_ref, b2v_ref,
               Wgt_ref, bgt_ref, W1t_ref, b1t_ref, W2t_ref, b2t_ref,
               Wto_ref, bto_ref,
               Wg1a_ref, Wg1b_ref, bg1_ref, Wg2_ref, bg2_ref,
               Wu1a_ref, Wu1b_ref, bu1_ref, Wu2_ref, bu2_ref,
               Wh_ref, bh_ref, rw_ref, rb_ref, mean_ref, std_ref,
               out_ref):
    xt = xt_ref[0]

    tokens = _dot(xt, We_ref[...]) + be_ref[...]
    xu = xt / (jnp.sqrt(jnp.sum(xt * xt, axis=-1, keepdims=True)) + 1e-8)
    S = jax.lax.dot_general(xu, xu, (((1,), (1,)), ((), ())),
                            preferred_element_type=jnp.float32)
    mask = _topk3_mask(S, axis=1)
    Av = _softmax(jnp.where(mask, S, NEG), axis=-1)
    h = tokens
    for l in range(EL):
        m = _dot(Av, h)
        g = jax.nn.gelu(_dot(m, Wgv_ref[l]) + bgv_ref[l][None, :])
        h = _ln(h + g)
        f = jax.nn.gelu(_dot(h, W1v_ref[l]) + b1v_ref[l][None, :])
        f = _dot(f, W2v_ref[l]) + b2v_ref[l][None, :]
        h = _ln(h + f)
    z_var = h

    hb = []
    tub = []
    for p in range(P):
        tp = _dot(xt[:, p * PL:(p + 1) * PL], Wt_ref[...]) + bt_ref[...]
        hb.append(tp)
        tub.append(tp / (jnp.sqrt(jnp.sum(tp * tp, axis=-1,
                                          keepdims=True)) + 1e-8))
    Arow = []
    for p in range(P):
        sp = jnp.concatenate(
            [jnp.sum(tub[p] * tub[q], axis=-1, keepdims=True)
             for q in range(P)], axis=1)
        mk = _topk3_mask(sp, axis=1)
        Arow.append(_softmax(jnp.where(mk, sp, NEG), axis=-1))
    for l in range(EL):
        mb = []
        for p in range(P):
            m = Arow[p][:, 0:1] * hb[0]
            for q in range(1, P):
                m = m + Arow[p][:, q:q + 1] * hb[q]
            mb.append(m)
        for p in range(P):
            g = jax.nn.gelu(_dot(mb[p], Wgt_ref[l]) + bgt_ref[l][None, :])
            hp = _ln(hb[p] + g)
            f = jax.nn.gelu(_dot(hp, W1t_ref[l]) + b1t_ref[l][None, :])
            f = _dot(f, W2t_ref[l]) + b2t_ref[l][None, :]
            hb[p] = _ln(hp + f)
    zm = hb[0]
    for p in range(1, P):
        zm = zm + hb[p]
    zm = zm * (1.0 / P)
    z_time = _dot(zm, Wto_ref[...]) + bto_ref[...]

    g1 = jax.nn.gelu(_dot(z_var, Wg1a_ref[...]) + _dot(z_time, Wg1b_ref[...])
                     + bg1_ref[...])
    wsm = _softmax(_dot(g1, Wg2_ref[...]) + bg2_ref[...], axis=-1)
    u = jax.nn.gelu(_dot(z_var, Wu1a_ref[...]) + _dot(z_time, Wu1b_ref[...])
                    + bu1_ref[...])
    u = _dot(u, Wu2_ref[...]) + bu2_ref[...]
    fused = wsm[:, 0:1] * z_var + wsm[:, 1:2] * z_time + 0.1 * u \
        + 0.2 * tokens
    y = _dot(fused, Wh_ref[...]) + bh_ref[...]
    out_ref[0] = (y - rb_ref[...]) / (rw_ref[...] + 1e-10) * std_ref[0] \
        + mean_ref[0]


def _full(shape):
    nd = len(shape)
    return pl.BlockSpec(shape, lambda b: (0,) * nd)


def _bat(shape):
    zeros = (0,) * len(shape)
    return pl.BlockSpec((1,) + shape, lambda b, z=zeros: (b,) + z)


def kernel(x_enc, revin_w, revin_b, W_emb, b_emb, W_tp, b_tp, Wg_var, bg_var,
           Wf1_var, bf1_var, Wf2_var, bf2_var, Wg_t, bg_t, Wf1_t, bf1_t,
           Wf2_t, bf2_t, W_to, b_to, Wg1, bg1, Wg2, bg2, Wu1, bu1, Wu2, bu2,
           W_h, b_h):
    f32 = jnp.float32
    nb = x_enc.shape[0]
    grid = (nb,)

    xn, mean, std = pl.pallas_call(
        _prep_body,
        grid=grid,
        compiler_params=_PAR,
        in_specs=[_bat((L, N)), _full((1, N)), _full((1, N))],
        out_specs=[_bat((L, N)), _bat((1, N)), _bat((1, N))],
        out_shape=[jax.ShapeDtypeStruct((nb, L, N), f32),
                   jax.ShapeDtypeStruct((nb, 1, N), f32),
                   jax.ShapeDtypeStruct((nb, 1, N), f32)],
    )(x_enc, revin_w.reshape(1, N), revin_b.reshape(1, N))

    xt = jnp.swapaxes(xn, 1, 2)
    stdT = jnp.swapaxes(std, 1, 2)
    meanT = jnp.swapaxes(mean, 1, 2)

    yk = pl.pallas_call(
        _mega_body,
        grid=grid,
        compiler_params=_PAR,
        in_specs=[_bat((N, L)), _full((L, D)), _full((1, D)),
                  _full((PL, D)), _full((1, D)),
                  _full((EL, D, D)), _full((EL, D)), _full((EL, D, DF)),
                  _full((EL, DF)), _full((EL, DF, D)), _full((EL, D)),
                  _full((EL, D, D)), _full((EL, D)), _full((EL, D, DF)),
                  _full((EL, DF)), _full((EL, DF, D)), _full((EL, D)),
                  _full((D, D)), _full((1, D)),
                  _full((D, D)), _full((D, D)), _full((1, D)),
                  _full((D, 2)), _full((1, 2)),
                  _full((D, D)), _full((D, D)), _full((1, D)),
                  _full((D, D)), _full((1, D)),
                  _full((D, H)), _full((1, H)),
                  _full((N, 1)), _full((N, 1)), _bat((N, 1)), _bat((N, 1))],
        out_specs=_bat((N, H)),
        out_shape=jax.ShapeDtypeStruct((nb, N, H), f32),
    )(xt, W_emb, b_emb.reshape(1, D), W_tp, b_tp.reshape(1, D),
      Wg_var, bg_var, Wf1_var, bf1_var, Wf2_var, bf2_var,
      Wg_t, bg_t, Wf1_t, bf1_t, Wf2_t, bf2_t,
      W_to, b_to.reshape(1, D),
      Wg1[:D], Wg1[D:], bg1.reshape(1, D), Wg2, bg2.reshape(1, 2),
      Wu1[:D], Wu1[D:], bu1.reshape(1, D), Wu2, bu2.reshape(1, D),
      W_h, b_h.reshape(1, H),
      revin_w.reshape(N, 1), revin_b.reshape(N, 1), meanT, stdT)

    return jnp.swapaxes(yk, 1, 2)


# speedup vs baseline: 6.3705x; 1.0167x over previous
import jax
import jax.numpy as jnp
from jax.experimental import pallas as pl
from jax.experimental.pallas import tpu as pltpu

B, L, N, D, DF, P, PL, H, K, EL = 16, 336, 321, 512, 512, 7, 48, 96, 3, 2
NEG = -1e9

_PAR = pltpu.CompilerParams(dimension_semantics=("parallel",))


def _ln(x):
    mu = x.mean(axis=-1, keepdims=True)
    var = x.var(axis=-1, keepdims=True)
    return (x - mu) / jnp.sqrt(var + 1e-5)


def _topk3_mask(S, axis):
    cols = jax.lax.broadcasted_iota(jnp.int32, S.shape, axis)
    Sm = S
    mask = jnp.zeros(S.shape, jnp.bool_)
    for _ in range(K):
        m = jnp.max(Sm, axis=axis, keepdims=True)
        c = jnp.min(jnp.where(Sm >= m, cols, jnp.int32(2**30)), axis=axis,
                    keepdims=True)
        hit = cols == c
        mask = jnp.logical_or(mask, hit)
        Sm = jnp.where(hit, NEG, Sm)
    return mask


def _softmax(x, axis=-1):
    m = jnp.max(x, axis=axis, keepdims=True)
    e = jnp.exp(x - m)
    return e / jnp.sum(e, axis=axis, keepdims=True)


def _dot(a, b):
    return jnp.dot(a, b, preferred_element_type=jnp.float32)


def _dotT(a, b):
    return jax.lax.dot_general(a, b, (((0,), (0,)), ((), ())),
                               preferred_element_type=jnp.float32)


def _mega_body(x_ref, rw_ref, rb_ref, We_ref, be_ref, Wt_ref, bt_ref,
               Wgv_ref, bgv_ref, W1v_ref, b1v_ref, W2v_ref, b2v_ref,
               Wgt_ref, bgt_ref, W1t_ref, b1t_ref, W2t_ref, b2t_ref,
               Wto_ref, bto_ref,
               Wg1a_ref, Wg1b_ref, bg1_ref, Wg2_ref, bg2_ref,
               Wu1a_ref, Wu1b_ref, bu1_ref, Wu2_ref, bu2_ref,
               Wh_ref, bh_ref, out_ref):
    x = x_ref[0]
    mean = jnp.mean(x, axis=0, keepdims=True)
    var = jnp.mean((x - mean) * (x - mean), axis=0, keepdims=True)
    std = jnp.sqrt(var + 1e-5)
    xn = (x - mean) / std * rw_ref[...] + rb_ref[...]

    tokens = _dotT(xn, We_ref[...]) + be_ref[...]
    xnu = xn / (jnp.sqrt(jnp.sum(xn * xn, axis=0, keepdims=True)) + 1e-8)
    S = _dotT(xnu, xnu)
    mask = _topk3_mask(S, axis=1)
    Av = _softmax(jnp.where(mask, S, NEG), axis=-1)
    h = tokens
    for l in range(EL):
        m = _dot(Av, h)
        g = jax.nn.gelu(_dot(m, Wgv_ref[l]) + bgv_ref[l][None, :])
        h = _ln(h + g)
        f = jax.nn.gelu(_dot(h, W1v_ref[l]) + b1v_ref[l][None, :])
        f = _dot(f, W2v_ref[l]) + b2v_ref[l][None, :]
        h = _ln(h + f)
    z_var = h

    hb = []
    tub = []
    for p in range(P):
        tp = _dotT(xn[p * PL:(p + 1) * PL, :], Wt_ref[...]) + bt_ref[...]
        hb.append(tp)
        tub.append(tp / (jnp.sqrt(jnp.sum(tp * tp, axis=-1,
                                          keepdims=True)) + 1e-8))
    Arow = []
    for p in range(P):
        sp = jnp.concatenate(
            [jnp.sum(tub[p] * tub[q], axis=-1, keepdims=True)
             for q in range(P)], axis=1)
        mk = _topk3_mask(sp, axis=1)
        Arow.append(_softmax(jnp.where(mk, sp, NEG), axis=-1))
    for l in range(EL):
        mb = []
        for p in range(P):
            m = Arow[p][:, 0:1] * hb[0]
            for q in range(1, P):
                m = m + Arow[p][:, q:q + 1] * hb[q]
            mb.append(m)
        for p in range(P):
            g = jax.nn.gelu(_dot(mb[p], Wgt_ref[l]) + bgt_ref[l][None, :])
            hp = _ln(hb[p] + g)
            f = jax.nn.gelu(_dot(hp, W1t_ref[l]) + b1t_ref[l][None, :])
            f = _dot(f, W2t_ref[l]) + b2t_ref[l][None, :]
            hb[p] = _ln(hp + f)
    zm = hb[0]
    for p in range(1, P):
        zm = zm + hb[p]
    zm = zm * (1.0 / P)
    z_time = _dot(zm, Wto_ref[...]) + bto_ref[...]

    g1 = jax.nn.gelu(_dot(z_var, Wg1a_ref[...]) + _dot(z_time, Wg1b_ref[...])
                     + bg1_ref[...])
    wsm = _softmax(_dot(g1, Wg2_ref[...]) + bg2_ref[...], axis=-1)
    u = jax.nn.gelu(_dot(z_var, Wu1a_ref[...]) + _dot(z_time, Wu1b_ref[...])
                    + bu1_ref[...])
    u = _dot(u, Wu2_ref[...]) + bu2_ref[...]
    fused = wsm[:, 0:1] * z_var + wsm[:, 1:2] * z_time + 0.1 * u \
        + 0.2 * tokens
    y = jax.lax.dot_general(Wh_ref[...], fused, (((0,), (1,)), ((), ())),
                            preferred_element_type=jnp.float32) \
        + bh_ref[...]
    out_ref[0] = (y - rb_ref[...]) / (rw_ref[...] + 1e-10) * std + mean


def _full(shape):
    nd = len(shape)
    return pl.BlockSpec(shape, lambda b: (0,) * nd)


def _bat(shape):
    zeros = (0,) * len(shape)
    return pl.BlockSpec((1,) + shape, lambda b, z=zeros: (b,) + z)


def kernel(x_enc, revin_w, revin_b, W_emb, b_emb, W_tp, b_tp, Wg_var, bg_var,
           Wf1_var, bf1_var, Wf2_var, bf2_var, Wg_t, bg_t, Wf1_t, bf1_t,
           Wf2_t, bf2_t, W_to, b_to, Wg1, bg1, Wg2, bg2, Wu1, bu1, Wu2, bu2,
           W_h, b_h):
    f32 = jnp.float32
    nb = x_enc.shape[0]

    y = pl.pallas_call(
        _mega_body,
        grid=(nb,),
        compiler_params=_PAR,
        in_specs=[_bat((L, N)), _full((1, N)), _full((1, N)),
                  _full((L, D)), _full((1, D)),
                  _full((PL, D)), _full((1, D)),
                  _full((EL, D, D)), _full((EL, D)), _full((EL, D, DF)),
                  _full((EL, DF)), _full((EL, DF, D)), _full((EL, D)),
                  _full((EL, D, D)), _full((EL, D)), _full((EL, D, DF)),
                  _full((EL, DF)), _full((EL, DF, D)), _full((EL, D)),
                  _full((D, D)), _full((1, D)),
                  _full((D, D)), _full((D, D)), _full((1, D)),
                  _full((D, 2)), _full((1, 2)),
                  _full((D, D)), _full((D, D)), _full((1, D)),
                  _full((D, D)), _full((1, D)),
                  _full((D, H)), _full((H, 1))],
        out_specs=_bat((H, N)),
        out_shape=jax.ShapeDtypeStruct((nb, H, N), f32),
    )(x_enc, revin_w.reshape(1, N), revin_b.reshape(1, N),
      W_emb, b_emb.reshape(1, D), W_tp, b_tp.reshape(1, D),
      Wg_var, bg_var, Wf1_var, bf1_var, Wf2_var, bf2_var,
      Wg_t, bg_t, Wf1_t, bf1_t, Wf2_t, bf2_t,
      W_to, b_to.reshape(1, D),
      Wg1[:D], Wg1[D:], bg1.reshape(1, D), Wg2, bg2.reshape(1, 2),
      Wu1[:D], Wu1[D:], bu1.reshape(1, D), Wu2, bu2.reshape(1, D),
      W_h, b_h.reshape(H, 1))

    return y
